# baseline (speedup 1.0000x reference)
import dataclasses
import functools

import jax
import jax.numpy as jnp
from jax import lax
from jax.experimental import pallas as pl
from jax.experimental.pallas import tpu as pltpu
from jax.experimental.pallas import tpu_sc as plsc

N = 10000
NP = 10240
D = 128
E = 320000

NC = 2
NS = 16
GROUP = 128
EP = 327680
GP_TOT = EP // GROUP
NG_SUB = GP_TOT // (NC * NS)
SUB_E = NG_SUB * GROUP
ROWS_PER_SUB = NP // NS
RCHUNK = 128

_mesh_cache = []


def _mesh():
    if not _mesh_cache:
        _mesh_cache.append(plsc.VectorSubcoreMesh(
            core_axis_name="c", subcore_axis_name="s",
            num_cores=NC, num_subcores=NS,
        ))
    return _mesh_cache[0]


def _sc_segsum(table, src2d, dst2d, zeros_block, with_counts=False):
    out_types = [jax.ShapeDtypeStruct((NC * NP, D), jnp.float32)]
    scratch = [
        pltpu.VMEM((NG_SUB, GROUP), jnp.int32),
        pltpu.VMEM((NG_SUB, GROUP), jnp.int32),
        pltpu.VMEM((GROUP, D), jnp.float32),
        pltpu.VMEM_SHARED((NP, D), jnp.float32),
        pltpu.SemaphoreType.DMA,
    ]
    cp = pltpu.CompilerParams()
    if with_counts:
        out_types.append(jax.ShapeDtypeStruct((NC * NS, SUB_E), jnp.float32))
        scratch.append(pltpu.VMEM((SUB_E,), jnp.float32))
        if "needs_layout_passes" in pltpu.CompilerParams.__dataclass_fields__:
            cp = dataclasses.replace(cp, needs_layout_passes=False)

    @functools.partial(
        pl.kernel,
        out_type=tuple(out_types) if with_counts else out_types[0],
        mesh=_mesh(),
        compiler_params=cp,
        scratch_types=scratch,
    )
    def k(table_hbm, src_hbm, dst_hbm, zero_hbm, *rest):
        if with_counts:
            out_hbm, cnt_hbm, sidx, didx, rows0, acc, sem0, cacc = rest
        else:
            out_hbm, sidx, didx, rows0, acc, sem0 = rest
        cid = lax.axis_index("c")
        sid = lax.axis_index("s")

        pltpu.sync_copy(zero_hbm, rows0)
        rbase = sid * ROWS_PER_SUB
        for j in range(ROWS_PER_SUB // RCHUNK):
            pltpu.sync_copy(
                rows0.at[pl.ds(0, RCHUNK)],
                acc.at[pl.ds(rbase + j * RCHUNK, RCHUNK)],
            )

        gbase = cid * (GP_TOT // NC) + sid * NG_SUB
        pltpu.sync_copy(src_hbm.at[pl.ds(gbase, NG_SUB)], sidx)
        pltpu.sync_copy(dst_hbm.at[pl.ds(gbase, NG_SUB)], didx)
        plsc.subcore_barrier()

        @pl.loop(0, NG_SUB, unroll=2)
        def _(j):
            pltpu.async_copy(table_hbm.at[sidx.at[j]], rows0, sem0).wait()
            pltpu.sync_copy(rows0, acc.at[didx.at[j]], add=True)

        if with_counts:
            wid = cid * NS + sid

            @pl.loop(0, SUB_E, step=16)
            def _(e):
                cacc[pl.ds(e, 16)] = jnp.zeros((16,), jnp.float32)

            nrows = jnp.minimum(jnp.maximum(E // GROUP - gbase, 0), NG_SUB)
            ones16 = jnp.ones((16,), jnp.float32)

            @pl.loop(0, nrows)
            def _(j):
                for kk in range(GROUP // 16):
                    iv = didx[j, pl.ds(kk * 16, 16)]
                    plsc.addupdate_scatter(cacc, [iv], ones16)

            pltpu.sync_copy(cacc, cnt_hbm.at[wid])

        plsc.subcore_barrier()
        pltpu.sync_copy(
            acc.at[pl.ds(rbase, ROWS_PER_SUB)],
            out_hbm.at[pl.ds(cid * NP + rbase, ROWS_PER_SUB)],
        )

    return k(table, src2d, dst2d, zeros_block)


_BLK = 1024
_GRID = NP // _BLK


def _tc_proj(x, WpT, bp):
    def body(x_ref, w_ref, b_ref, o_ref):
        o_ref[...] = jax.nn.relu(
            jnp.dot(x_ref[...], w_ref[...],
                    preferred_element_type=jnp.float32) + b_ref[...]
        )

        @pl.when(pl.program_id(0) == _GRID - 1)
        def _():
            o_ref[pl.ds(_BLK - (NP - N), NP - N), :] = jnp.zeros(
                (NP - N, D), jnp.float32)

    return pl.pallas_call(
        body,
        grid=(_GRID,),
        in_specs=[
            pl.BlockSpec((_BLK, D), lambda i: (i, 0)),
            pl.BlockSpec((D, D), lambda i: (0, 0)),
            pl.BlockSpec((1, D), lambda i: (0, 0)),
        ],
        out_specs=pl.BlockSpec((_BLK, D), lambda i: (i, 0)),
        out_shape=jax.ShapeDtypeStruct((NP, D), jnp.float32),
    )(x, WpT, bp.reshape(1, D))


def _tc_post(partials, cnt32, x, WlT, bl, WrT, norm_relu, WpT=None, bp=None):
    fuse_proj = WpT is not None
    npad = NP - N

    def body(*refs):
        if fuse_proj:
            p_ref, c_ref, x_ref, wl_ref, b_ref, wr_ref, wp_ref, bp_ref, \
                o_ref, xs_ref = refs
        else:
            p_ref, c_ref, x_ref, wl_ref, b_ref, wr_ref, o_ref = refs
        summed = p_ref[0] + p_ref[1]
        cvec = jnp.sum(c_ref[...], axis=0)
        eye = jnp.eye(128, dtype=jnp.float32)
        cols = [
            jnp.sum(eye * cvec[b * 128:(b + 1) * 128], axis=1, keepdims=True)
            for b in range(8)
        ]
        cnt = jnp.concatenate(cols, axis=0)
        mean = summed / jnp.maximum(cnt, 1.0)
        out = (
            jnp.dot(mean, wl_ref[...], preferred_element_type=jnp.float32)
            + b_ref[...]
            + jnp.dot(x_ref[...], wr_ref[...],
                      preferred_element_type=jnp.float32)
        )
        if norm_relu:
            nrm = jnp.sqrt(jnp.sum(out * out, axis=-1, keepdims=True))
            out = jax.nn.relu(out / jnp.maximum(nrm, 1e-12))
        o_ref[...] = out
        if fuse_proj:
            xs_ref[...] = jax.nn.relu(
                jnp.dot(out, wp_ref[...],
                        preferred_element_type=jnp.float32) + bp_ref[...])

        @pl.when(pl.program_id(0) == _GRID - 1)
        def _():
            zpad = jnp.zeros((npad, D), jnp.float32)
            o_ref[pl.ds(_BLK - npad, npad), :] = zpad
            if fuse_proj:
                xs_ref[pl.ds(_BLK - npad, npad), :] = zpad

    in_specs = [
        pl.BlockSpec((NC, _BLK, D), lambda i: (0, i, 0)),
        pl.BlockSpec((NC * NS, _BLK), lambda i: (0, i)),
        pl.BlockSpec((_BLK, D), lambda i: (i, 0)),
        pl.BlockSpec((D, D), lambda i: (0, 0)),
        pl.BlockSpec((1, D), lambda i: (0, 0)),
        pl.BlockSpec((D, D), lambda i: (0, 0)),
    ]
    args = [partials, cnt32, x, WlT, bl.reshape(1, D), WrT]
    out_shape = jax.ShapeDtypeStruct((NP, D), jnp.float32)
    out_specs = pl.BlockSpec((_BLK, D), lambda i: (i, 0))
    if fuse_proj:
        in_specs += [pl.BlockSpec((D, D), lambda i: (0, 0)),
                     pl.BlockSpec((1, D), lambda i: (0, 0))]
        args += [WpT, bp.reshape(1, D)]
        return pl.pallas_call(
            body, grid=(_GRID,), in_specs=in_specs,
            out_specs=(out_specs, out_specs),
            out_shape=(out_shape, out_shape),
        )(*args)
    return pl.pallas_call(
        body, grid=(_GRID,), in_specs=in_specs,
        out_specs=out_specs, out_shape=out_shape,
    )(*args)


def kernel(x, edge_index, c0_Wp, c0_bp, c0_Wl, c0_bl, c0_Wr,
           c1_Wp, c1_bp, c1_Wl, c1_bl, c1_Wr, o_Wl, o_bl, o_Wr):
    src = edge_index[0]
    dst = edge_index[1]
    pad_iota = jnp.arange(EP - E, dtype=jnp.int32)
    src2d = jnp.concatenate(
        [src, N + pad_iota % (NP - N)]).reshape(GP_TOT, GROUP)
    dst_pad = jnp.concatenate([dst, pad_iota % NP])
    dst2d = dst_pad.reshape(GP_TOT, GROUP)

    zrows = jnp.zeros((GROUP, D), jnp.float32)
    x_p = jnp.concatenate([x, jnp.zeros((NP - N, D), jnp.float32)])

    xs0 = _tc_proj(x_p, c0_Wp.T, c0_bp)
    p0, cnt32 = _sc_segsum(xs0, src2d, dst2d, zrows, with_counts=True)
    p0 = p0.reshape(NC, NP, D)
    h1, xs1 = _tc_post(p0, cnt32, x_p, c0_Wl.T, c0_bl, c0_Wr.T, True,
                       c1_Wp.T, c1_bp)
    p1 = _sc_segsum(xs1, src2d, dst2d, zrows).reshape(NC, NP, D)
    h2 = _tc_post(p1, cnt32, h1, c1_Wl.T, c1_bl, c1_Wr.T, True)

    p2 = _sc_segsum(h2, src2d, dst2d, zrows).reshape(NC, NP, D)
    out = _tc_post(p2, cnt32, h2, o_Wl.T, o_bl, o_Wr.T, False)
    return out[:N]


# speedup vs baseline: 1.0021x; 1.0021x over previous
import dataclasses
import functools

import jax
import jax.numpy as jnp
from jax import lax
from jax.experimental import pallas as pl
from jax.experimental.pallas import tpu as pltpu
from jax.experimental.pallas import tpu_sc as plsc

N = 10000
NP = 10240
D = 128
E = 320000

NC = 2
NS = 16
GROUP = 128
EP = 327680
GP_TOT = EP // GROUP
NG_SUB = GP_TOT // (NC * NS)
SUB_E = NG_SUB * GROUP
ROWS_PER_SUB = NP // NS
RCHUNK = 128

_mesh_cache = []


def _mesh():
    if not _mesh_cache:
        _mesh_cache.append(plsc.VectorSubcoreMesh(
            core_axis_name="c", subcore_axis_name="s",
            num_cores=NC, num_subcores=NS,
        ))
    return _mesh_cache[0]


def _sc_segsum(table, src2d, dst2d, zeros_block, with_counts=False):
    out_types = [jax.ShapeDtypeStruct((NC * NP, D), jnp.float32)]
    scratch = [
        pltpu.VMEM((NG_SUB, GROUP), jnp.int32),
        pltpu.VMEM((NG_SUB, GROUP), jnp.int32),
        pltpu.VMEM((GROUP, D), jnp.float32),
        pltpu.VMEM_SHARED((NP, D), jnp.float32),
        pltpu.SemaphoreType.DMA,
    ]
    cp = pltpu.CompilerParams()
    if with_counts:
        out_types.append(jax.ShapeDtypeStruct((NC * NS, SUB_E), jnp.float32))
        scratch.append(pltpu.VMEM((SUB_E,), jnp.float32))
        if "needs_layout_passes" in pltpu.CompilerParams.__dataclass_fields__:
            cp = dataclasses.replace(cp, needs_layout_passes=False)

    @functools.partial(
        pl.kernel,
        out_type=tuple(out_types) if with_counts else out_types[0],
        mesh=_mesh(),
        compiler_params=cp,
        scratch_types=scratch,
    )
    def k(table_hbm, src_hbm, dst_hbm, zero_hbm, *rest):
        if with_counts:
            out_hbm, cnt_hbm, sidx, didx, rows0, acc, sem0, cacc = rest
        else:
            out_hbm, sidx, didx, rows0, acc, sem0 = rest
        cid = lax.axis_index("c")
        sid = lax.axis_index("s")

        pltpu.sync_copy(zero_hbm, rows0)
        rbase = sid * ROWS_PER_SUB
        for j in range(ROWS_PER_SUB // RCHUNK):
            pltpu.sync_copy(
                rows0.at[pl.ds(0, RCHUNK)],
                acc.at[pl.ds(rbase + j * RCHUNK, RCHUNK)],
            )

        gbase = cid * (GP_TOT // NC) + sid * NG_SUB
        pltpu.sync_copy(src_hbm.at[pl.ds(gbase, NG_SUB)], sidx)
        pltpu.sync_copy(dst_hbm.at[pl.ds(gbase, NG_SUB)], didx)
        plsc.subcore_barrier()

        @pl.loop(0, NG_SUB)
        def _(j):
            pltpu.async_copy(table_hbm.at[sidx.at[j]], rows0, sem0).wait()
            pltpu.sync_copy(rows0, acc.at[didx.at[j]], add=True)

        if with_counts:
            wid = cid * NS + sid

            @pl.loop(0, SUB_E, step=16)
            def _(e):
                cacc[pl.ds(e, 16)] = jnp.zeros((16,), jnp.float32)

            nrows = jnp.minimum(jnp.maximum(E // GROUP - gbase, 0), NG_SUB)
            ones16 = jnp.ones((16,), jnp.float32)

            @pl.loop(0, nrows)
            def _(j):
                for kk in range(GROUP // 16):
                    iv = didx[j, pl.ds(kk * 16, 16)]
                    plsc.addupdate_scatter(cacc, [iv], ones16)

            pltpu.sync_copy(cacc, cnt_hbm.at[wid])

        plsc.subcore_barrier()
        pltpu.sync_copy(
            acc.at[pl.ds(rbase, ROWS_PER_SUB)],
            out_hbm.at[pl.ds(cid * NP + rbase, ROWS_PER_SUB)],
        )

    return k(table, src2d, dst2d, zeros_block)


_BLK = 1024
_GRID = NP // _BLK


def _tc_proj(x, WpT, bp):
    def body(x_ref, w_ref, b_ref, o_ref):
        o_ref[...] = jax.nn.relu(
            jnp.dot(x_ref[...], w_ref[...],
                    preferred_element_type=jnp.float32) + b_ref[...]
        )

        @pl.when(pl.program_id(0) == _GRID - 1)
        def _():
            o_ref[pl.ds(_BLK - (NP - N), NP - N), :] = jnp.zeros(
                (NP - N, D), jnp.float32)

    return pl.pallas_call(
        body,
        grid=(_GRID,),
        in_specs=[
            pl.BlockSpec((_BLK, D), lambda i: (i, 0)),
            pl.BlockSpec((D, D), lambda i: (0, 0)),
            pl.BlockSpec((1, D), lambda i: (0, 0)),
        ],
        out_specs=pl.BlockSpec((_BLK, D), lambda i: (i, 0)),
        out_shape=jax.ShapeDtypeStruct((NP, D), jnp.float32),
    )(x, WpT, bp.reshape(1, D))


def _tc_post(partials, cnt32, x, WlT, bl, WrT, norm_relu, WpT=None, bp=None):
    fuse_proj = WpT is not None
    npad = NP - N

    def body(*refs):
        if fuse_proj:
            p_ref, c_ref, x_ref, wl_ref, b_ref, wr_ref, wp_ref, bp_ref, \
                o_ref, xs_ref = refs
        else:
            p_ref, c_ref, x_ref, wl_ref, b_ref, wr_ref, o_ref = refs
        summed = p_ref[0] + p_ref[1]
        cvec = jnp.sum(c_ref[...], axis=0)
        eye = jnp.eye(128, dtype=jnp.float32)
        cols = [
            jnp.sum(eye * cvec[b * 128:(b + 1) * 128], axis=1, keepdims=True)
            for b in range(8)
        ]
        cnt = jnp.concatenate(cols, axis=0)
        mean = summed / jnp.maximum(cnt, 1.0)
        out = (
            jnp.dot(mean, wl_ref[...], preferred_element_type=jnp.float32)
            + b_ref[...]
            + jnp.dot(x_ref[...], wr_ref[...],
                      preferred_element_type=jnp.float32)
        )
        if norm_relu:
            nrm = jnp.sqrt(jnp.sum(out * out, axis=-1, keepdims=True))
            out = jax.nn.relu(out / jnp.maximum(nrm, 1e-12))
        o_ref[...] = out
        if fuse_proj:
            xs_ref[...] = jax.nn.relu(
                jnp.dot(out, wp_ref[...],
                        preferred_element_type=jnp.float32) + bp_ref[...])

        @pl.when(pl.program_id(0) == _GRID - 1)
        def _():
            zpad = jnp.zeros((npad, D), jnp.float32)
            o_ref[pl.ds(_BLK - npad, npad), :] = zpad
            if fuse_proj:
                xs_ref[pl.ds(_BLK - npad, npad), :] = zpad

    in_specs = [
        pl.BlockSpec((NC, _BLK, D), lambda i: (0, i, 0)),
        pl.BlockSpec((NC * NS, _BLK), lambda i: (0, i)),
        pl.BlockSpec((_BLK, D), lambda i: (i, 0)),
        pl.BlockSpec((D, D), lambda i: (0, 0)),
        pl.BlockSpec((1, D), lambda i: (0, 0)),
        pl.BlockSpec((D, D), lambda i: (0, 0)),
    ]
    args = [partials, cnt32, x, WlT, bl.reshape(1, D), WrT]
    out_shape = jax.ShapeDtypeStruct((NP, D), jnp.float32)
    out_specs = pl.BlockSpec((_BLK, D), lambda i: (i, 0))
    if fuse_proj:
        in_specs += [pl.BlockSpec((D, D), lambda i: (0, 0)),
                     pl.BlockSpec((1, D), lambda i: (0, 0))]
        args += [WpT, bp.reshape(1, D)]
        return pl.pallas_call(
            body, grid=(_GRID,), in_specs=in_specs,
            out_specs=(out_specs, out_specs),
            out_shape=(out_shape, out_shape),
        )(*args)
    return pl.pallas_call(
        body, grid=(_GRID,), in_specs=in_specs,
        out_specs=out_specs, out_shape=out_shape,
    )(*args)


def kernel(x, edge_index, c0_Wp, c0_bp, c0_Wl, c0_bl, c0_Wr,
           c1_Wp, c1_bp, c1_Wl, c1_bl, c1_Wr, o_Wl, o_bl, o_Wr):
    src = edge_index[0]
    dst = edge_index[1]
    pad_iota = jnp.arange(EP - E, dtype=jnp.int32)
    src2d = jnp.concatenate(
        [src, N + pad_iota % (NP - N)]).reshape(GP_TOT, GROUP)
    dst_pad = jnp.concatenate([dst, pad_iota % NP])
    dst2d = dst_pad.reshape(GP_TOT, GROUP)

    zrows = jnp.zeros((GROUP, D), jnp.float32)
    x_p = jnp.concatenate([x, jnp.zeros((NP - N, D), jnp.float32)])

    xs0 = _tc_proj(x_p, c0_Wp.T, c0_bp)
    p0, cnt32 = _sc_segsum(xs0, src2d, dst2d, zrows, with_counts=True)
    p0 = p0.reshape(NC, NP, D)
    h1, xs1 = _tc_post(p0, cnt32, x_p, c0_Wl.T, c0_bl, c0_Wr.T, True,
                       c1_Wp.T, c1_bp)
    p1 = _sc_segsum(xs1, src2d, dst2d, zrows).reshape(NC, NP, D)
    h2 = _tc_post(p1, cnt32, h1, c1_Wl.T, c1_bl, c1_Wr.T, True)

    p2 = _sc_segsum(h2, src2d, dst2d, zrows).reshape(NC, NP, D)
    out = _tc_post(p2, cnt32, h2, o_Wl.T, o_bl, o_Wr.T, False)
    return out[:N]
